# baseline (speedup 1.0000x reference)
import functools

import jax
import jax.numpy as jnp
from jax import lax
from jax.experimental import pallas as pl
from jax.experimental.pallas import tpu as pltpu
from jax.experimental.pallas import tpu_sc as plsc

_INFO = plsc.get_sparse_core_info()
_NC = _INFO.num_cores
_NS = _INFO.num_subcores
_NW = _NC * _NS

_CHUNK = 32
_NBUF = 3


@functools.cache
def _make_copy(n_rows: int, dim: int):
    rows_per_w = n_rows // _NW
    n_chunks = rows_per_w // _CHUNK
    mesh = plsc.VectorSubcoreMesh(core_axis_name="c", subcore_axis_name="s")

    @functools.partial(
        pl.kernel,
        out_type=jax.ShapeDtypeStruct((n_rows, dim), jnp.float32),
        mesh=mesh,
        scratch_types=[
            pltpu.VMEM((_NBUF, _CHUNK, dim), jnp.float32),
            pltpu.SemaphoreType.DMA,
            pltpu.SemaphoreType.DMA,
        ],
    )
    def copy_kernel(table_hbm, out_hbm, buf, ld_sem, st_sem):
        wid = lax.axis_index("s") * _NC + lax.axis_index("c")
        base = wid * rows_per_w

        def load(c):
            return pltpu.make_async_copy(
                table_hbm.at[pl.ds(base + c * _CHUNK, _CHUNK)],
                buf.at[lax.rem(c, _NBUF) if not isinstance(c, int) else c % _NBUF],
                ld_sem,
            )

        def store(c):
            return pltpu.make_async_copy(
                buf.at[lax.rem(c, _NBUF) if not isinstance(c, int) else c % _NBUF],
                out_hbm.at[pl.ds(base + c * _CHUNK, _CHUNK)],
                st_sem,
            )

        load(0).start()
        load(1).start()

        def body(c, carry):
            @pl.when(jnp.logical_and(c >= 1, c + 2 < n_chunks))
            def _():
                store(c - 1).wait()

            @pl.when(c + 2 < n_chunks)
            def _():
                load(c + 2).start()

            load(c).wait()
            store(c).start()
            return carry

        lax.fori_loop(0, n_chunks, body, 0)
        for _ in range(min(_NBUF, n_chunks)):
            store(0).wait()

    return copy_kernel


def kernel(x, position_embeddings):
    n_rows = x.shape[1]
    dim = position_embeddings.shape[1]
    return _make_copy(n_rows, dim)(position_embeddings)
